# Initial kernel scaffold; baseline (speedup 1.0000x reference)
import jax
import jax.numpy as jnp
from jax.experimental import pallas as pl


def kernel(driverID, weekID, timeID, dist, W_driver, W_week, W_time):
    raise NotImplementedError("write your pallas kernel here")


# speedup vs baseline: 2.8799x; 2.8799x over previous
import functools

import jax
import jax.numpy as jnp
from jax import lax
from jax.experimental import pallas as pl
from jax.experimental.pallas import tpu as pltpu
from jax.experimental.pallas import tpu_sc as plsc

B = 16384
D_DRV, D_WEEK, D_TIME = 16, 3, 8
D_OUT = D_DRV + D_WEEK + D_TIME + 1
V_DRV, V_WEEK, V_TIME = 24000, 7, 1440
SUP = 128
RPS = SUP // D_DRV
NC, NS = 2, 16
NW = NC * NS
BPW = B // NW
CHUNK = 128
NCHUNK = BPW // CHUNK
L = 16
NGROUP = BPW // L

_mesh = plsc.VectorSubcoreMesh(core_axis_name="c", subcore_axis_name="s")


@functools.partial(
    pl.kernel,
    mesh=_mesh,
    out_type=jax.ShapeDtypeStruct((B * D_OUT,), jnp.float32),
    compiler_params=pltpu.CompilerParams(needs_layout_passes=False),
    scratch_types=[
        pltpu.VMEM((BPW,), jnp.int32),
        pltpu.VMEM((BPW,), jnp.int32),
        pltpu.VMEM((BPW,), jnp.int32),
        pltpu.VMEM((BPW,), jnp.int32),
        pltpu.VMEM((BPW,), jnp.float32),
        pltpu.VMEM((BPW, SUP), jnp.float32),
        pltpu.VMEM((V_WEEK * D_WEEK,), jnp.float32),
        pltpu.VMEM((V_TIME * D_TIME,), jnp.float32),
        pltpu.VMEM((BPW * D_OUT,), jnp.float32),
        pltpu.SemaphoreType.DMA,
    ],
)
def _embed_concat(drv_id, week_id, time_id, dist, w_drv_sup, w_week, w_time,
                  out, drv_idx, drv_sup, week_idx, time_idx, dist_v,
                  g_buf, week_tbl, time_tbl, out_buf, sem):
    wid = lax.axis_index("s") * NC + lax.axis_index("c")
    base = wid * BPW
    pltpu.sync_copy(drv_id.at[pl.ds(base, BPW)], drv_idx)

    iota = lax.iota(jnp.int32, L)

    def sup_body(g, carry):
        v = drv_idx[pl.ds(g * L, L)]
        drv_sup[pl.ds(g * L, L)] = lax.shift_right_logical(v, 3)
        return carry

    lax.fori_loop(0, NGROUP, sup_body, 0)

    gathers = [
        pltpu.async_copy(w_drv_sup.at[drv_sup.at[pl.ds(j * CHUNK, CHUNK)]],
                         g_buf.at[pl.ds(j * CHUNK, CHUNK)], sem)
        for j in range(NCHUNK)
    ]

    pltpu.sync_copy(week_id.at[pl.ds(base, BPW)], week_idx)
    pltpu.sync_copy(time_id.at[pl.ds(base, BPW)], time_idx)
    pltpu.sync_copy(dist.at[pl.ds(base, BPW)], dist_v)
    pltpu.sync_copy(w_week, week_tbl)
    pltpu.sync_copy(w_time, time_tbl)

    def small_body(g, carry):
        rows = g * L + iota
        obase = rows * D_OUT
        widx = week_idx[pl.ds(g * L, L)] * D_WEEK
        tidx = time_idx[pl.ds(g * L, L)] * D_TIME
        for j in range(D_WEEK):
            v = plsc.load_gather(week_tbl, [widx + j])
            plsc.store_scatter(out_buf, [obase + (D_DRV + j)], v)
        for j in range(D_TIME):
            v = plsc.load_gather(time_tbl, [tidx + j])
            plsc.store_scatter(out_buf, [obase + (D_DRV + D_WEEK + j)], v)
        dvals = dist_v[pl.ds(g * L, L)]
        plsc.store_scatter(out_buf, [obase + (D_OUT - 1)], dvals)
        return carry

    lax.fori_loop(0, NGROUP, small_body, 0)
    for g in gathers:
        g.wait()

    def drv_body(g, carry):
        rows = g * L + iota
        obase = rows * D_OUT
        off = (drv_idx[pl.ds(g * L, L)] & (RPS - 1)) * D_DRV
        for j in range(D_DRV):
            v = plsc.load_gather(g_buf, [rows, off + j])
            plsc.store_scatter(out_buf, [obase + j], v)
        return carry

    lax.fori_loop(0, NGROUP, drv_body, 0)
    pltpu.sync_copy(out_buf, out.at[pl.ds(base * D_OUT, BPW * D_OUT)])


def kernel(driverID, weekID, timeID, dist, W_driver, W_week, W_time):
    flat = _embed_concat(driverID.astype(jnp.int32), weekID.astype(jnp.int32),
                         timeID.astype(jnp.int32), dist.astype(jnp.float32),
                         W_driver.reshape(V_DRV * D_DRV // SUP, SUP),
                         W_week.reshape(-1), W_time.reshape(-1))
    return flat.reshape(B, D_OUT)


# speedup vs baseline: 2.9431x; 1.0219x over previous
import functools

import jax
import jax.numpy as jnp
from jax import lax
from jax.experimental import pallas as pl
from jax.experimental.pallas import tpu as pltpu
from jax.experimental.pallas import tpu_sc as plsc

B = 16384
D_DRV, D_WEEK, D_TIME = 16, 3, 8
D_OUT = D_DRV + D_WEEK + D_TIME + 1
V_DRV, V_WEEK, V_TIME = 24000, 7, 1440
SUP = 128
RPS = SUP // D_DRV
NC, NS = 2, 16
NW = NC * NS
BPW = B // NW
CHUNK = 128
NCHUNK = BPW // CHUNK
L = 16
GPC = CHUNK // L

_mesh = plsc.VectorSubcoreMesh(core_axis_name="c", subcore_axis_name="s")


@functools.partial(
    pl.kernel,
    mesh=_mesh,
    out_type=jax.ShapeDtypeStruct((B, D_OUT), jnp.float32),
    compiler_params=pltpu.CompilerParams(needs_layout_passes=False),
    scratch_types=[
        pltpu.VMEM((BPW,), jnp.int32),
        pltpu.VMEM((BPW,), jnp.int32),
        pltpu.VMEM((BPW,), jnp.int32),
        pltpu.VMEM((BPW,), jnp.int32),
        pltpu.VMEM((BPW,), jnp.float32),
        pltpu.VMEM((BPW, SUP), jnp.float32),
        pltpu.VMEM((V_WEEK * D_WEEK,), jnp.float32),
        pltpu.VMEM((V_TIME * D_TIME,), jnp.float32),
        pltpu.VMEM((2, CHUNK, D_OUT), jnp.float32),
        [pltpu.SemaphoreType.DMA] * NCHUNK,
        [pltpu.SemaphoreType.DMA] * 2,
    ],
)
def _embed_concat(drv_id, week_id, time_id, dist, w_drv_sup, w_week, w_time,
                  out, drv_idx, drv_sup, week_idx, time_idx, dist_v,
                  g_buf, week_tbl, time_tbl, out_buf, gsems, osems):
    wid = lax.axis_index("s") * NC + lax.axis_index("c")
    base = wid * BPW
    pltpu.sync_copy(drv_id.at[pl.ds(base, BPW)], drv_idx)

    iota = lax.iota(jnp.int32, L)
    cols = [jnp.full((L,), c, jnp.int32) for c in range(D_OUT)]

    def sup_body(g, carry):
        v = drv_idx[pl.ds(g * L, L)]
        drv_sup[pl.ds(g * L, L)] = lax.shift_right_logical(v, 3)
        return carry

    lax.fori_loop(0, BPW // L, sup_body, 0)

    gathers = [
        pltpu.async_copy(w_drv_sup.at[drv_sup.at[pl.ds(j * CHUNK, CHUNK)]],
                         g_buf.at[pl.ds(j * CHUNK, CHUNK)], gsems[j])
        for j in range(NCHUNK)
    ]

    pltpu.sync_copy(week_id.at[pl.ds(base, BPW)], week_idx)
    pltpu.sync_copy(time_id.at[pl.ds(base, BPW)], time_idx)
    pltpu.sync_copy(dist.at[pl.ds(base, BPW)], dist_v)
    pltpu.sync_copy(w_week, week_tbl)
    pltpu.sync_copy(w_time, time_tbl)

    out_writes = [None, None]
    for r in range(NCHUNK):
        gathers[r].wait()
        buf = out_buf.at[r % 2]
        if out_writes[r % 2] is not None:
            out_writes[r % 2].wait()
        for g in range(GPC):
            rows = g * L + iota
            src = r * CHUNK + g * L
            dv = drv_idx[pl.ds(src, L)]
            off = (dv & (RPS - 1)) * D_DRV
            for j in range(D_DRV):
                v = plsc.load_gather(g_buf, [src + iota, off + j])
                plsc.store_scatter(buf, [rows, cols[j]], v)
            widx = week_idx[pl.ds(src, L)] * D_WEEK
            for j in range(D_WEEK):
                v = plsc.load_gather(week_tbl, [widx + j])
                plsc.store_scatter(buf, [rows, cols[D_DRV + j]], v)
            tidx = time_idx[pl.ds(src, L)] * D_TIME
            for j in range(D_TIME):
                v = plsc.load_gather(time_tbl, [tidx + j])
                plsc.store_scatter(buf, [rows, cols[D_DRV + D_WEEK + j]], v)
            dvals = dist_v[pl.ds(src, L)]
            plsc.store_scatter(buf, [rows, cols[D_OUT - 1]], dvals)
        out_writes[r % 2] = pltpu.async_copy(
            buf, out.at[pl.ds(base + r * CHUNK, CHUNK)], osems[r % 2])
    for w in out_writes:
        if w is not None:
            w.wait()


def kernel(driverID, weekID, timeID, dist, W_driver, W_week, W_time):
    return _embed_concat(driverID.astype(jnp.int32), weekID.astype(jnp.int32),
                         timeID.astype(jnp.int32), dist.astype(jnp.float32),
                         W_driver.reshape(V_DRV * D_DRV // SUP, SUP),
                         W_week.reshape(-1), W_time.reshape(-1))


# speedup vs baseline: 3.0128x; 1.0237x over previous
import functools

import jax
import jax.numpy as jnp
from jax import lax
from jax.experimental import pallas as pl
from jax.experimental.pallas import tpu as pltpu
from jax.experimental.pallas import tpu_sc as plsc

B = 16384
D_DRV, D_WEEK, D_TIME = 16, 3, 8
D_OUT = D_DRV + D_WEEK + D_TIME + 1
V_DRV, V_WEEK, V_TIME = 24000, 7, 1440
SUP = 128
RPS = SUP // D_DRV
NC, NS = 2, 16
NW = NC * NS
BPW = B // NW
CHUNK = 128
NCHUNK = BPW // CHUNK
L = 16
GPC = CHUNK // L

_mesh = plsc.VectorSubcoreMesh(core_axis_name="c", subcore_axis_name="s")


@functools.partial(
    pl.kernel,
    mesh=_mesh,
    out_type=jax.ShapeDtypeStruct((B, D_OUT), jnp.float32),
    compiler_params=pltpu.CompilerParams(needs_layout_passes=False),
    scratch_types=[
        pltpu.VMEM((BPW,), jnp.int32),
        pltpu.VMEM((BPW,), jnp.int32),
        pltpu.VMEM((BPW,), jnp.int32),
        pltpu.VMEM((BPW,), jnp.int32),
        pltpu.VMEM((BPW,), jnp.float32),
        pltpu.VMEM((BPW, SUP), jnp.float32),
        pltpu.VMEM((V_WEEK * D_WEEK,), jnp.float32),
        pltpu.VMEM((V_TIME * D_TIME,), jnp.float32),
        pltpu.VMEM((2, CHUNK, D_OUT), jnp.float32),
        [pltpu.SemaphoreType.DMA] * NCHUNK,
        [pltpu.SemaphoreType.DMA] * 2,
        pltpu.SemaphoreType.DMA,
    ],
)
def _embed_concat(drv_id, week_id, time_id, dist, w_drv_sup, w_week, w_time,
                  out, drv_idx, drv_sup, week_idx, time_idx, dist_v,
                  g_buf, week_tbl, time_tbl, out_buf, gsems, osems, ssem):
    wid = lax.axis_index("s") * NC + lax.axis_index("c")
    base = wid * BPW
    pltpu.sync_copy(drv_id.at[pl.ds(base, BPW)], drv_idx)

    iota = lax.iota(jnp.int32, L)
    cols = [jnp.full((L,), c, jnp.int32) for c in range(D_OUT)]

    def sup_body(g, carry):
        v = drv_idx[pl.ds(g * L, L)]
        drv_sup[pl.ds(g * L, L)] = lax.shift_right_logical(v, 3)
        return carry

    lax.fori_loop(0, BPW // L, sup_body, 0)

    gathers = [
        pltpu.async_copy(w_drv_sup.at[drv_sup.at[pl.ds(j * CHUNK, CHUNK)]],
                         g_buf.at[pl.ds(j * CHUNK, CHUNK)], gsems[j])
        for j in range(NCHUNK)
    ]

    stagers = [
        pltpu.async_copy(week_id.at[pl.ds(base, BPW)], week_idx, ssem),
        pltpu.async_copy(time_id.at[pl.ds(base, BPW)], time_idx, ssem),
        pltpu.async_copy(dist.at[pl.ds(base, BPW)], dist_v, ssem),
        pltpu.async_copy(w_week, week_tbl, ssem),
        pltpu.async_copy(w_time, time_tbl, ssem),
    ]
    for s in stagers:
        s.wait()

    out_writes = [None, None]
    for r in range(NCHUNK):
        buf = out_buf.at[r % 2]
        if out_writes[r % 2] is not None:
            out_writes[r % 2].wait()
        for g in range(GPC):
            rows = g * L + iota
            src = r * CHUNK + g * L
            widx = week_idx[pl.ds(src, L)] * D_WEEK
            for j in range(D_WEEK):
                v = plsc.load_gather(week_tbl, [widx + j])
                plsc.store_scatter(buf, [rows, cols[D_DRV + j]], v)
            tidx = time_idx[pl.ds(src, L)] * D_TIME
            for j in range(D_TIME):
                v = plsc.load_gather(time_tbl, [tidx + j])
                plsc.store_scatter(buf, [rows, cols[D_DRV + D_WEEK + j]], v)
            dvals = dist_v[pl.ds(src, L)]
            plsc.store_scatter(buf, [rows, cols[D_OUT - 1]], dvals)
        gathers[r].wait()
        for g in range(GPC):
            rows = g * L + iota
            src = r * CHUNK + g * L
            dv = drv_idx[pl.ds(src, L)]
            off = (dv & (RPS - 1)) * D_DRV
            for j in range(D_DRV):
                v = plsc.load_gather(g_buf, [src + iota, off + j])
                plsc.store_scatter(buf, [rows, cols[j]], v)
        out_writes[r % 2] = pltpu.async_copy(
            buf, out.at[pl.ds(base + r * CHUNK, CHUNK)], osems[r % 2])
    for w in out_writes:
        if w is not None:
            w.wait()


def kernel(driverID, weekID, timeID, dist, W_driver, W_week, W_time):
    return _embed_concat(driverID.astype(jnp.int32), weekID.astype(jnp.int32),
                         timeID.astype(jnp.int32), dist.astype(jnp.float32),
                         W_driver.reshape(V_DRV * D_DRV // SUP, SUP),
                         W_week.reshape(-1), W_time.reshape(-1))


# speedup vs baseline: 3.7128x; 1.2323x over previous
import functools

import jax
import jax.numpy as jnp
from jax import lax
from jax.experimental import pallas as pl
from jax.experimental.pallas import tpu as pltpu
from jax.experimental.pallas import tpu_sc as plsc

B = 16384
D_DRV, D_WEEK, D_TIME = 16, 3, 8
D_OUT = D_DRV + D_WEEK + D_TIME + 1
V_DRV, V_WEEK, V_TIME = 24000, 7, 1440
SUP = 128
RPS = SUP // D_DRV
NC, NS = 2, 16
NW = NC * NS
BPW = B // NW
CHUNK = 128
NCHUNK = BPW // CHUNK
L = 16
GPC = CHUNK // L

_mesh = plsc.VectorSubcoreMesh(core_axis_name="c", subcore_axis_name="s")


@functools.partial(
    pl.kernel,
    mesh=_mesh,
    out_type=jax.ShapeDtypeStruct((D_OUT, B), jnp.float32),
    compiler_params=pltpu.CompilerParams(needs_layout_passes=False),
    scratch_types=[
        pltpu.VMEM((BPW,), jnp.int32),
        pltpu.VMEM((BPW,), jnp.int32),
        pltpu.VMEM((BPW,), jnp.int32),
        pltpu.VMEM((BPW,), jnp.int32),
        pltpu.VMEM((BPW,), jnp.float32),
        pltpu.VMEM((BPW, SUP), jnp.float32),
        pltpu.VMEM((V_WEEK * D_WEEK,), jnp.float32),
        pltpu.VMEM((V_TIME * D_TIME,), jnp.float32),
        pltpu.VMEM((2, D_OUT, CHUNK), jnp.float32),
        [pltpu.SemaphoreType.DMA] * NCHUNK,
        [pltpu.SemaphoreType.DMA] * 2,
        pltpu.SemaphoreType.DMA,
    ],
)
def _embed_concat(drv_id, week_id, time_id, dist, w_drv_sup, w_week, w_time,
                  out, drv_idx, drv_sup, week_idx, time_idx, dist_v,
                  g_buf, week_tbl, time_tbl, out_buf, gsems, osems, ssem):
    wid = lax.axis_index("s") * NC + lax.axis_index("c")
    base = wid * BPW
    pltpu.sync_copy(drv_id.at[pl.ds(base, BPW)], drv_idx)

    iota = lax.iota(jnp.int32, L)
    cols = [jnp.full((L,), c, jnp.int32) for c in range(D_OUT)]

    def sup_body(g, carry):
        v = drv_idx[pl.ds(g * L, L)]
        drv_sup[pl.ds(g * L, L)] = lax.shift_right_logical(v, 3)
        return carry

    lax.fori_loop(0, BPW // L, sup_body, 0)

    gathers = [
        pltpu.async_copy(w_drv_sup.at[drv_sup.at[pl.ds(j * CHUNK, CHUNK)]],
                         g_buf.at[pl.ds(j * CHUNK, CHUNK)], gsems[j])
        for j in range(NCHUNK)
    ]

    stagers = [
        pltpu.async_copy(week_id.at[pl.ds(base, BPW)], week_idx, ssem),
        pltpu.async_copy(time_id.at[pl.ds(base, BPW)], time_idx, ssem),
        pltpu.async_copy(dist.at[pl.ds(base, BPW)], dist_v, ssem),
        pltpu.async_copy(w_week, week_tbl, ssem),
        pltpu.async_copy(w_time, time_tbl, ssem),
    ]
    for s in stagers:
        s.wait()

    out_writes = [None, None]
    for r in range(NCHUNK):
        buf = out_buf.at[r % 2]
        if out_writes[r % 2] is not None:
            out_writes[r % 2].wait()
        for g in range(GPC):
            rows = g * L + iota
            src = r * CHUNK + g * L
            widx = week_idx[pl.ds(src, L)] * D_WEEK
            for j in range(D_WEEK):
                v = plsc.load_gather(week_tbl, [widx + j])
                plsc.store_scatter(buf, [cols[D_DRV + j], rows], v)
            tidx = time_idx[pl.ds(src, L)] * D_TIME
            for j in range(D_TIME):
                v = plsc.load_gather(time_tbl, [tidx + j])
                plsc.store_scatter(buf, [cols[D_DRV + D_WEEK + j], rows], v)
            dvals = dist_v[pl.ds(src, L)]
            plsc.store_scatter(buf, [cols[D_OUT - 1], rows], dvals)
        gathers[r].wait()
        for g in range(GPC):
            rows = g * L + iota
            src = r * CHUNK + g * L
            dv = drv_idx[pl.ds(src, L)]
            off = (dv & (RPS - 1)) * D_DRV
            for j in range(D_DRV):
                v = plsc.load_gather(g_buf, [src + iota, off + j])
                plsc.store_scatter(buf, [cols[j], rows], v)
        out_writes[r % 2] = pltpu.async_copy(
            buf, out.at[:, pl.ds(base + r * CHUNK, CHUNK)], osems[r % 2])
    for w in out_writes:
        if w is not None:
            w.wait()


def kernel(driverID, weekID, timeID, dist, W_driver, W_week, W_time):
    w_flat = lax.optimization_barrier(W_driver.reshape(-1))
    out_t = _embed_concat(driverID.astype(jnp.int32), weekID.astype(jnp.int32),
                          timeID.astype(jnp.int32), dist.astype(jnp.float32),
                          w_flat.reshape(V_DRV * D_DRV // SUP, SUP),
                          W_week.reshape(-1), W_time.reshape(-1))
    return out_t.T
